# baseline (speedup 1.0000x reference)
import jax
import jax.numpy as jnp
from jax.experimental import pallas as pl
from jax.experimental.pallas import tpu as pltpu

_B = 1024
_L = 20
_H = 128


def _knot_body(br_ref, init_ref, th_ref, g_ref, bt_ref,
               w1_ref, b1_ref, w2_ref, b2_ref, w3_ref, b3_ref,
               o1_ref, o2_ref, m_ref, w1_vmem, w1_sem):
    f32 = jnp.float32
    w1_copy = pltpu.make_async_copy(w1_ref, w1_vmem, w1_sem)
    w1_copy.start()
    braidsT = jnp.transpose(br_ref[...])
    iot = jax.lax.broadcasted_iota(jnp.int32, (32, _B), 0)
    m_ref[...] = jnp.where((iot % 4) == (iot // 8), 1.0, 0.0).astype(f32)
    trig = []
    for ppp in range(3):
        cs = []
        for fn in (jnp.cos, jnp.sin):
            rows = [jnp.broadcast_to(fn(th_ref[l:l + 1, ppp:ppp + 1]), (4, 1))
                    for l in range(2)]
            cs.append(jnp.concatenate(rows, axis=0))
        trig.append(cs)
    for t in range(_L):
        gen = braidsT[t:t + 1, :]
        sgn = jnp.where(gen > 0, 1.0, -1.0).astype(f32)
        absg = jnp.abs(gen)
        for ppp in range(3):
            active = absg == (ppp + 1)
            c8, s8 = trig[ppp]
            c = jnp.where(active, c8, 1.0)
            s = jnp.where(active, sgn * s8, 0.0)
            u = m_ref[pl.ds(8 * ppp, 8), :]
            v = m_ref[pl.ds(8 * ppp + 8, 8), :]
            m_ref[pl.ds(8 * ppp, 8), :] = c * u - s * v
            m_ref[pl.ds(8 * ppp + 8, 8), :] = s * u + c * v
    mm = m_ref[...]
    initT = jnp.transpose(init_ref[...])
    gT = jnp.transpose(g_ref[...])
    btT = jnp.transpose(bt_ref[...])
    prev = None
    for layer in range(2):
        news = []
        for i in range(4):
            acc = None
            for j in range(4):
                r = i * 8 + layer * 4 + j
                mrow = mm[r:r + 1, :]
                col = initT[:, j:j + 1] if layer == 0 else prev[j]
                term = col * mrow
                acc = term if acc is None else acc + term
            news.append(acc)
        gcol = gT[:, layer:layer + 1]
        bcol = btT[:, layer:layer + 1]
        prev = []
        for i in range(4):
            x = news[i]
            mean = jnp.mean(x, axis=0, keepdims=True)
            var = jnp.mean((x - mean) ** 2, axis=0, keepdims=True)
            prev.append((x - mean) / jnp.sqrt(var + 1e-5) * gcol + bcol)
    b1c = jnp.transpose(jnp.reshape(b1_ref[...], (1, 128)))
    b2c = jnp.transpose(jnp.reshape(b2_ref[...], (1, 64)))
    b3c = jnp.transpose(jnp.reshape(b3_ref[...], (1, 2)))
    flat = jnp.concatenate(prev, axis=0)
    w1_copy.wait()
    h1 = jnp.dot(w1_vmem[...], flat, preferred_element_type=f32) + b1c
    h1 = jnp.maximum(h1, 0.0)
    h2 = jnp.dot(w2_ref[...], h1, preferred_element_type=f32) + b2c
    h2 = jnp.maximum(h2, 0.0)
    out = jnp.dot(w3_ref[...], h2, preferred_element_type=f32) + b3c
    o1_ref[...] = jnp.reshape(jax.nn.sigmoid(out[0:1, :]), (_B,))
    o2_ref[...] = jnp.reshape(out[1:2, :], (_B,))


def kernel(braids, initial_state, thetas, ln_gamma, ln_beta,
           w1, b1, w2, b2, w3, b3):
    n_in = 11
    in_specs = [pl.BlockSpec(memory_space=pl.ANY) if i == 5
                else pl.BlockSpec(memory_space=pltpu.MemorySpace.VMEM)
                for i in range(n_in)]
    o1, o2 = pl.pallas_call(
        _knot_body,
        out_shape=[jax.ShapeDtypeStruct((_B,), jnp.float32),
                   jax.ShapeDtypeStruct((_B,), jnp.float32)],
        in_specs=in_specs,
        scratch_shapes=[pltpu.VMEM((32, _B), jnp.float32),
                        pltpu.VMEM((128, 512), jnp.float32),
                        pltpu.SemaphoreType.DMA],
    )(braids, initial_state, thetas, ln_gamma, ln_beta,
      w1, b1, w2, b2, w3, b3)
    return o1, o2


# speedup vs baseline: 1.1174x; 1.1174x over previous
import jax
import jax.numpy as jnp
from jax.experimental import pallas as pl
from jax.experimental.pallas import tpu as pltpu

_B = 1024
_L = 20
_H = 128


def _knot_body(br_ref, init_ref, th_ref, g_ref, bt_ref,
               w1_ref, b1_ref, w2_ref, b2_ref, w3_ref, b3_ref,
               o1_ref, o2_ref, m_ref):
    f32 = jnp.float32
    braidsT = jnp.transpose(br_ref[...])
    iot = jax.lax.broadcasted_iota(jnp.int32, (32, _B), 0)
    m_ref[...] = jnp.where((iot % 4) == (iot // 8), 1.0, 0.0).astype(f32)
    trig = []
    for ppp in range(3):
        cs = []
        for fn in (jnp.cos, jnp.sin):
            rows = [jnp.broadcast_to(fn(th_ref[l:l + 1, ppp:ppp + 1]), (4, 1))
                    for l in range(2)]
            cs.append(jnp.concatenate(rows, axis=0))
        trig.append(cs)
    for t in range(_L):
        gen = braidsT[t:t + 1, :]
        sgn = jnp.where(gen > 0, 1.0, -1.0).astype(f32)
        absg = jnp.abs(gen)
        for ppp in range(3):
            active = absg == (ppp + 1)
            c8, s8 = trig[ppp]
            c = jnp.where(active, c8, 1.0)
            s = jnp.where(active, sgn * s8, 0.0)
            u = m_ref[pl.ds(8 * ppp, 8), :]
            v = m_ref[pl.ds(8 * ppp + 8, 8), :]
            m_ref[pl.ds(8 * ppp, 8), :] = c * u - s * v
            m_ref[pl.ds(8 * ppp + 8, 8), :] = s * u + c * v
    mm = m_ref[...]
    initT = jnp.transpose(init_ref[...])
    gT = jnp.transpose(g_ref[...])
    btT = jnp.transpose(bt_ref[...])
    ones_r = jnp.ones((1, _H), f32)
    inv_h = 1.0 / _H
    rows0 = [jnp.concatenate([mm[i * 8 + j:i * 8 + j + 1, :]
                              for i in range(4)], axis=1) for j in range(4)]
    mmat0 = jnp.concatenate(rows0, axis=0)
    x0 = jnp.dot(initT, mmat0, preferred_element_type=f32)
    sm = jnp.dot(ones_r, x0, preferred_element_type=f32) * inv_h
    sq = jnp.dot(ones_r, x0 * x0, preferred_element_type=f32) * inv_h
    inv = jax.lax.rsqrt(sq - sm * sm + 1e-5)
    prev_all = (x0 - sm) * inv * gT[:, 0:1] + btT[:, 0:1]
    prev = [prev_all[:, i * _B:(i + 1) * _B] for i in range(4)]
    g1 = gT[:, 1:2]
    bt1 = btT[:, 1:2]
    prev1 = []
    for i in range(4):
        acc = None
        for j in range(4):
            term = prev[j] * mm[i * 8 + 4 + j:i * 8 + 5 + j, :]
            acc = term if acc is None else acc + term
        sm_i = jnp.dot(ones_r, acc, preferred_element_type=f32) * inv_h
        sq_i = jnp.dot(ones_r, acc * acc, preferred_element_type=f32) * inv_h
        inv_i = jax.lax.rsqrt(sq_i - sm_i * sm_i + 1e-5)
        prev1.append((acc - sm_i) * inv_i * g1 + bt1)
    b1c = jnp.transpose(jnp.reshape(b1_ref[...], (1, 128)))
    b2c = jnp.transpose(jnp.reshape(b2_ref[...], (1, 64)))
    b3c = jnp.transpose(jnp.reshape(b3_ref[...], (1, 2)))
    h1 = b1c
    for i in range(4):
        h1 = h1 + jnp.dot(w1_ref[:, i * _H:(i + 1) * _H], prev1[i],
                          preferred_element_type=f32)
    h1 = jnp.maximum(h1, 0.0)
    h2 = jnp.dot(w2_ref[...], h1, preferred_element_type=f32) + b2c
    h2 = jnp.maximum(h2, 0.0)
    out = jnp.dot(w3_ref[...], h2, preferred_element_type=f32) + b3c
    o1_ref[...] = jnp.reshape(jax.nn.sigmoid(out[0:1, :]), (_B,))
    o2_ref[...] = jnp.reshape(out[1:2, :], (_B,))


def kernel(braids, initial_state, thetas, ln_gamma, ln_beta,
           w1, b1, w2, b2, w3, b3):
    o1, o2 = pl.pallas_call(
        _knot_body,
        out_shape=[jax.ShapeDtypeStruct((_B,), jnp.float32),
                   jax.ShapeDtypeStruct((_B,), jnp.float32)],
        scratch_shapes=[pltpu.VMEM((32, _B), jnp.float32)],
    )(braids, initial_state, thetas, ln_gamma, ln_beta,
      w1, b1, w2, b2, w3, b3)
    return o1, o2
